# baseline (speedup 1.0000x reference)
import jax
import jax.numpy as jnp
from jax import lax
from jax.experimental import pallas as pl
from jax.experimental.pallas import tpu as pltpu
from jax.experimental.pallas import tpu_sc as plsc

_T = 8192
_BC = 8
_NQ = 4
_TCH = _T // _NQ
_G = 16
_PCOLS = _G + _TCH + _G
_OUT = _T * 8 + 8
_OUTPAD = _T * 8 + 128
_CONTRIB = _TCH * 8 + 128


def _decoder_body(est, out, tail, buf0, buf1, pbuf, contrib, halo, htmp,
                  sem0, sem1):
    c = lax.axis_index("c")
    s = lax.axis_index("s")
    w = c * 16 + s
    bc = w // _NQ
    q = w % _NQ
    t0 = q * _TCH

    zero16 = jnp.zeros((16,), jnp.float32)
    lane = lax.iota(jnp.int32, 16)
    rlow = lane & 7
    rhigh = rlow + 8
    colpat = lane >> 3

    for j in range(16):
        pbuf[j, pl.ds(0, _G)] = zero16
        pbuf[j, pl.ds(_G + _TCH, _G)] = zero16
    for jj in range(8):
        htmp[jj, :] = zero16

    @pl.when(q > 0)
    def _():
        for half in range(2):
            pltpu.sync_copy(
                est.at[bc, pl.ds(128 + 64 * half, 64), pl.ds(t0 - 128, 128)],
                halo)
            for jj in range(4):
                v = [halo[16 * jj + l, pl.ds(112, 16)] for l in range(16)]
                a = [v[2 * i] + v[2 * i + 1] for i in range(8)]
                b = [a[2 * i] + a[2 * i + 1] for i in range(4)]
                htmp[4 * half + jj, :] = (
                    (b[0] + b[1]) + (b[2] + b[3])) * (1.0 / 16.0)

    bufs = (buf0, buf1)
    sems = (sem0, sem1)
    copies = [None] * 16
    copies[0] = pltpu.async_copy(
        est.at[bc, pl.ds(0, 16), pl.ds(t0, _TCH)], bufs[0], sems[0])
    for j in range(16):
        if j + 1 < 16:
            copies[j + 1] = pltpu.async_copy(
                est.at[bc, pl.ds(16 * (j + 1), 16), pl.ds(t0, _TCH)],
                bufs[(j + 1) % 2], sems[(j + 1) % 2])
        copies[j].wait()
        cur = bufs[j % 2]

        @plsc.parallel_loop(0, _TCH // 16, unroll=4)
        def tt_body(tt, cur=cur, j=j):
            off = tt * 16
            v = [cur[l, pl.ds(off, 16)] for l in range(16)]
            a = [v[2 * i] + v[2 * i + 1] for i in range(8)]
            b = [a[2 * i] + a[2 * i + 1] for i in range(4)]
            c01 = b[0] + b[1]
            c23 = b[2] + b[3]
            pbuf[j, pl.ds(_G + off, 16)] = (c01 + c23) * (1.0 / 16.0)

    @plsc.parallel_loop(0, _TCH // 2 + 1, unroll=4)
    def pair_body(h):
        cola = colpat + (_G + 2 * h)
        ta = plsc.load_gather(pbuf, [rlow, cola])
        tb = plsc.load_gather(pbuf, [rhigh, cola - 1])
        contrib[pl.ds(h * 16, 16)] = ta + tb

    @pl.when(q > 0)
    def _():
        col15 = jnp.full((16,), 15, jnp.int32)
        hv = plsc.load_gather(htmp, [rlow, col15])
        hv = jnp.where(lane < 8, hv, 0.0)
        contrib[pl.ds(0, 16)] = contrib[pl.ds(0, 16)] + hv

    pltpu.sync_copy(contrib.at[pl.ds(0, _TCH * 8)],
                    out.at[bc, pl.ds(t0 * 8, _TCH * 8)])

    @pl.when(q == _NQ - 1)
    def _():
        pltpu.sync_copy(contrib.at[pl.ds(_TCH * 8, 128)], tail.at[bc])


def kernel(est_source):
    est = est_source.reshape(_BC, 256, _T)
    mesh = plsc.VectorSubcoreMesh(core_axis_name="c", subcore_axis_name="s")
    out = pl.kernel(
        _decoder_body,
        out_type=(jax.ShapeDtypeStruct((_BC, _OUT), jnp.float32),
                  jax.ShapeDtypeStruct((_BC, 128), jnp.float32)),
        mesh=mesh,
        scratch_types=[
            pltpu.VMEM((16, _TCH), jnp.float32),
            pltpu.VMEM((16, _TCH), jnp.float32),
            pltpu.VMEM((16, _PCOLS), jnp.float32),
            pltpu.VMEM((_CONTRIB,), jnp.float32),
            pltpu.VMEM((64, 128), jnp.float32),
            pltpu.VMEM((8, 16), jnp.float32),
            pltpu.SemaphoreType.DMA,
            pltpu.SemaphoreType.DMA,
        ],
        compiler_params=pltpu.CompilerParams(
            needs_layout_passes=False, skip_device_barrier=True),
    )(est)
    main, tail = out
    y = jax.lax.dynamic_update_slice(main, tail[:, :8], (0, _T * 8))
    return y.reshape(4, 2, _OUT)


# speedup vs baseline: 1.0637x; 1.0637x over previous
import jax
import jax.numpy as jnp
from jax import lax
from jax.experimental import pallas as pl
from jax.experimental.pallas import tpu as pltpu
from jax.experimental.pallas import tpu_sc as plsc

_T = 8192
_BC = 8
_NQ = 4
_TCH = _T // _NQ
_OUT = _T * 8 + 8
_OUTPAD = _T * 8 + 128
_CONTRIB = _TCH * 8 + 128
_PC = 128 + _TCH + 16


def _pool_body(x_ref, o_ref):
    x = x_ref[0]
    o_ref[0] = x.reshape(16, 16, x.shape[-1]).sum(axis=1) * (1.0 / 16.0)


def _oa_body(pooled, out, pbuf, contrib):
    c = lax.axis_index("c")
    s = lax.axis_index("s")
    w = c * 16 + s
    bc = w // _NQ
    q = w % _NQ
    t0 = q * _TCH

    zero16 = jnp.zeros((16,), jnp.float32)
    lane = lax.iota(jnp.int32, 16)
    rlow = lane & 7
    rhigh = rlow + 8
    colpat = lane >> 3

    for j in range(16):
        pbuf[j, pl.ds(112, 16)] = zero16
        pbuf[j, pl.ds(128 + _TCH, 16)] = zero16

    @pl.when(q > 0)
    def _():
        pltpu.sync_copy(pooled.at[bc, :, pl.ds(t0 - 128, 128 + _TCH)],
                        pbuf.at[:, pl.ds(0, 128 + _TCH)])

    @pl.when(q == 0)
    def _():
        pltpu.sync_copy(pooled.at[bc, :, pl.ds(0, _TCH)],
                        pbuf.at[:, pl.ds(128, _TCH)])

    @plsc.parallel_loop(0, _TCH // 2 + 1, unroll=4)
    def pair_body(h):
        cola = colpat + (128 + 2 * h)
        ta = plsc.load_gather(pbuf, [rlow, cola])
        tb = plsc.load_gather(pbuf, [rhigh, cola - 1])
        contrib[pl.ds(h * 16, 16)] = ta + tb

    pltpu.sync_copy(contrib.at[pl.ds(0, _TCH * 8)],
                    out.at[bc, pl.ds(t0 * 8, _TCH * 8)])

    @pl.when(q == _NQ - 1)
    def _():
        pltpu.sync_copy(contrib.at[pl.ds(_TCH * 8, 128)],
                        out.at[bc, pl.ds(_T * 8, 128)])


def kernel(est_source):
    est = est_source.reshape(_BC, 256, _T)

    pooled = pl.pallas_call(
        _pool_body,
        out_shape=jax.ShapeDtypeStruct((_BC, 16, _T), jnp.float32),
        grid=(_BC, 4),
        in_specs=[pl.BlockSpec((1, 256, _T // 4), lambda b, t: (b, 0, t))],
        out_specs=pl.BlockSpec((1, 16, _T // 4), lambda b, t: (b, 0, t)),
        compiler_params=pltpu.CompilerParams(
            dimension_semantics=("parallel", "parallel")),
    )(est)

    mesh = plsc.VectorSubcoreMesh(core_axis_name="c", subcore_axis_name="s")
    out = pl.kernel(
        _oa_body,
        out_type=jax.ShapeDtypeStruct((_BC, _OUTPAD), jnp.float32),
        mesh=mesh,
        scratch_types=[
            pltpu.VMEM((16, _PC), jnp.float32),
            pltpu.VMEM((_CONTRIB,), jnp.float32),
        ],
        compiler_params=pltpu.CompilerParams(needs_layout_passes=False),
    )(pooled)
    return out[:, :_OUT].reshape(4, 2, _OUT)


# speedup vs baseline: 1.2722x; 1.1960x over previous
import jax
import jax.numpy as jnp
from jax import lax
from jax.experimental import pallas as pl
from jax.experimental.pallas import tpu as pltpu
from jax.experimental.pallas import tpu_sc as plsc

_T = 8192
_BC = 8
_NQ = 4
_TCH = _T // _NQ
_OUT = _T * 8 + 8
_OUTPAD = _T * 8 + 128
_CONTRIB = _TCH * 8 + 128
_PC = 128 + _TCH + 16


def _pool_body(x_ref, o_ref):
    x = x_ref[0]
    o_ref[0] = x.reshape(16, 16, x.shape[-1]).sum(axis=1) * (1.0 / 16.0)


def _oa_body(pooled, out, pbuf, contrib):
    c = lax.axis_index("c")
    s = lax.axis_index("s")
    w = c * 16 + s
    bc = w // _NQ
    q = w % _NQ
    t0 = q * _TCH

    zero16 = jnp.zeros((16,), jnp.float32)
    lane = lax.iota(jnp.int32, 16)
    rlow = lane & 7
    rhigh = rlow + 8
    colpat = lane >> 3

    for j in range(16):
        pbuf[j, pl.ds(112, 16)] = zero16
        pbuf[j, pl.ds(128 + _TCH, 16)] = zero16

    @pl.when(q > 0)
    def _():
        pltpu.sync_copy(pooled.at[bc, :, pl.ds(t0 - 128, 128 + _TCH)],
                        pbuf.at[:, pl.ds(0, 128 + _TCH)])

    @pl.when(q == 0)
    def _():
        pltpu.sync_copy(pooled.at[bc, :, pl.ds(0, _TCH)],
                        pbuf.at[:, pl.ds(128, _TCH)])

    @plsc.parallel_loop(0, _TCH // 2 + 1, unroll=4)
    def pair_body(h):
        cola = colpat + (128 + 2 * h)
        ta = plsc.load_gather(pbuf, [rlow, cola])
        tb = plsc.load_gather(pbuf, [rhigh, cola - 1])
        contrib[pl.ds(h * 16, 16)] = ta + tb

    pltpu.sync_copy(contrib.at[pl.ds(0, _TCH * 8)],
                    out.at[bc, pl.ds(t0 * 8, _TCH * 8)])

    @pl.when(q == _NQ - 1)
    def _():
        pltpu.sync_copy(contrib.at[pl.ds(_TCH * 8, 128)],
                        out.at[bc, pl.ds(_T * 8, 128)])


def kernel(est_source):
    est = est_source.reshape(_BC, 256, _T)

    pooled = pl.pallas_call(
        _pool_body,
        out_shape=jax.ShapeDtypeStruct((_BC, 16, _T), jnp.float32),
        grid=(_BC,),
        in_specs=[pl.BlockSpec((1, 256, _T), lambda b: (b, 0, 0))],
        out_specs=pl.BlockSpec((1, 16, _T), lambda b: (b, 0, 0)),
        compiler_params=pltpu.CompilerParams(
            dimension_semantics=("parallel",)),
    )(est)

    mesh = plsc.VectorSubcoreMesh(core_axis_name="c", subcore_axis_name="s")
    out = pl.kernel(
        _oa_body,
        out_type=jax.ShapeDtypeStruct((_BC, _OUTPAD), jnp.float32),
        mesh=mesh,
        scratch_types=[
            pltpu.VMEM((16, _PC), jnp.float32),
            pltpu.VMEM((_CONTRIB,), jnp.float32),
        ],
        compiler_params=pltpu.CompilerParams(needs_layout_passes=False),
    )(pooled)
    return out[:, :_OUT].reshape(4, 2, _OUT)


# speedup vs baseline: 1.3034x; 1.0245x over previous
import jax
import jax.numpy as jnp
from jax import lax
from jax.experimental import pallas as pl
from jax.experimental.pallas import tpu as pltpu
from jax.experimental.pallas import tpu_sc as plsc

_T = 8192
_BC = 8
_NQ = 4
_TCH = _T // _NQ
_OUT = _T * 8 + 8
_OUTPAD = _T * 8 + 128
_CONTRIB = _TCH * 8 + 128
_PC = 128 + _TCH + 16


def _pool_body(x_ref, o_ref):
    x = x_ref[0]
    o_ref[0] = x.reshape(16, 16, x.shape[-1]).sum(axis=1) * (1.0 / 16.0)


def _oa_body(pooled, out, pbuf, contrib):
    c = lax.axis_index("c")
    s = lax.axis_index("s")
    w = c * 16 + s
    bc = w // _NQ
    q = w % _NQ
    t0 = q * _TCH

    zero16 = jnp.zeros((16,), jnp.float32)
    lane = lax.iota(jnp.int32, 16)
    rlow = lane & 7
    rhigh = rlow + 8
    colpat = lane >> 3

    for j in range(16):
        pbuf[j, pl.ds(112, 16)] = zero16
        pbuf[j, pl.ds(128 + _TCH, 16)] = zero16

    @pl.when(q > 0)
    def _():
        pltpu.sync_copy(pooled.at[bc, :, pl.ds(t0 - 128, 128 + _TCH)],
                        pbuf.at[:, pl.ds(0, 128 + _TCH)])

    @pl.when(q == 0)
    def _():
        pltpu.sync_copy(pooled.at[bc, :, pl.ds(0, _TCH)],
                        pbuf.at[:, pl.ds(128, _TCH)])

    @plsc.parallel_loop(0, _TCH // 2 + 1, unroll=8)
    def pair_body(h):
        cola = colpat + (128 + 2 * h)
        ta = plsc.load_gather(pbuf, [rlow, cola])
        tb = plsc.load_gather(pbuf, [rhigh, cola - 1])
        contrib[pl.ds(h * 16, 16)] = ta + tb

    pltpu.sync_copy(contrib.at[pl.ds(0, _TCH * 8)],
                    out.at[bc, pl.ds(t0 * 8, _TCH * 8)])

    @pl.when(q == _NQ - 1)
    def _():
        pltpu.sync_copy(contrib.at[pl.ds(_TCH * 8, 128)],
                        out.at[bc, pl.ds(_T * 8, 128)])


def kernel(est_source):
    est = est_source.reshape(_BC, 256, _T)

    pooled = pl.pallas_call(
        _pool_body,
        out_shape=jax.ShapeDtypeStruct((_BC, 16, _T), jnp.float32),
        grid=(_BC,),
        in_specs=[pl.BlockSpec((1, 256, _T), lambda b: (b, 0, 0))],
        out_specs=pl.BlockSpec((1, 16, _T), lambda b: (b, 0, 0)),
        compiler_params=pltpu.CompilerParams(
            dimension_semantics=("parallel",)),
    )(est)

    mesh = plsc.VectorSubcoreMesh(core_axis_name="c", subcore_axis_name="s")
    out = pl.kernel(
        _oa_body,
        out_type=jax.ShapeDtypeStruct((_BC, _OUTPAD), jnp.float32),
        mesh=mesh,
        scratch_types=[
            pltpu.VMEM((16, _PC), jnp.float32),
            pltpu.VMEM((_CONTRIB,), jnp.float32),
        ],
        compiler_params=pltpu.CompilerParams(needs_layout_passes=False),
    )(pooled)
    return out[:, :_OUT].reshape(4, 2, _OUT)
